# baseline (speedup 1.0000x reference)
import functools

import jax
import jax.numpy as jnp
from jax import lax
from jax.experimental import pallas as pl
from jax.experimental.pallas import tpu as pltpu
from jax.experimental.pallas import tpu_sc as plsc

N_NODES = 10000
N_EDGES = 320000
D = 128

NC = 2
NS = 16
NW = NC * NS

E_PER_W = N_EDGES // NW
CHUNK = 32
N_CHUNKS = -(-E_PER_W // CHUNK)
E_PAD = N_CHUNKS * CHUNK - E_PER_W
NR = 8
NG = NR - 2
NI = 12
UNROLL = 24

PAD_ROW = N_NODES
N_ACC = 10008

ROWS_PER_TILE = 624
ZERO_TAIL = N_ACC - NS * ROWS_PER_TILE
OUT_TAIL = N_NODES - NS * ROWS_PER_TILE


def _sc_segment_sum(x, ei4, zeros):
    mesh = plsc.VectorSubcoreMesh(core_axis_name="c", subcore_axis_name="s")

    @functools.partial(
        pl.kernel,
        out_type=jax.ShapeDtypeStruct((NC, N_NODES, D), jnp.float32),
        mesh=mesh,
        scratch_types=(
            [pltpu.VMEM_SHARED((N_ACC, D), jnp.float32)]
            + [pltpu.VMEM((2, CHUNK), jnp.int32)] * NI
            + [pltpu.VMEM((CHUNK, D), jnp.float32)] * NR
            + [pltpu.SemaphoreType.DMA] * (NI + 2 * NR + 1)
        ),
    )
    def body(x_hbm, ei_hbm, z_hbm, out_hbm, agg_sh, *scr):
        ibs = scr[:NI]
        rbs = scr[NI:NI + NR]
        sis = scr[NI + NR:2 * NI + NR]
        sgs = scr[2 * NI + NR:2 * NI + 2 * NR]
        sss = scr[2 * NI + 2 * NR:2 * NI + 3 * NR]
        sz = scr[-1]
        c = lax.axis_index("c")
        s = lax.axis_index("s")
        wid = c * NS + s

        row0 = s * ROWS_PER_TILE
        pltpu.async_copy(z_hbm.at[pl.ds(row0, ROWS_PER_TILE), :],
                         agg_sh.at[pl.ds(row0, ROWS_PER_TILE), :], sz)

        @pl.when(s == NS - 1)
        def _zero_tail():
            pltpu.sync_copy(z_hbm.at[pl.ds(NS * ROWS_PER_TILE, ZERO_TAIL), :],
                            agg_sh.at[pl.ds(NS * ROWS_PER_TILE, ZERO_TAIL), :])

        def idx_start(i, j):
            @pl.when(i < N_CHUNKS)
            def _():
                pltpu.async_copy(ei_hbm.at[wid, i], ibs[j], sis[j])

        def idx_wait(j):
            pltpu.make_async_copy(ei_hbm.at[wid, 0], ibs[j], sis[j]).wait()

        def gather_start(j, b):
            pltpu.async_copy(x_hbm.at[ibs[j].at[0]], rbs[b], sgs[b])

        def gather_wait(j, b):
            pltpu.make_async_copy(x_hbm.at[ibs[j].at[0]], rbs[b],
                                  sgs[b]).wait()

        def scatter_start(j, b):
            pltpu.async_copy(rbs[b], agg_sh.at[ibs[j].at[1]], sss[b],
                             add=True)

        def scatter_wait(j, b):
            pltpu.make_async_copy(rbs[b], agg_sh.at[ibs[j].at[1]],
                                  sss[b]).wait()

        for m in range(NG + 4):
            idx_start(m, m % NI)
        for m in range(NG):
            idx_wait(m % NI)
            gather_start(m % NI, m % NR)

        pltpu.make_async_copy(z_hbm.at[pl.ds(row0, ROWS_PER_TILE), :],
                              agg_sh.at[pl.ds(row0, ROWS_PER_TILE), :],
                              sz).wait()
        plsc.subcore_barrier()

        def ring_step(q, carry):
            i0 = q * UNROLL
            for k in range(UNROLL):
                i = i0 + k
                b, j = k % NR, k % NI
                bd, jd = (k + NR - 2) % NR, (k + NI - 2) % NI
                bg, jg = (k + NG) % NR, (k + NG) % NI

                @pl.when(i < N_CHUNKS)
                def _cur():
                    gather_wait(j, b)
                    scatter_start(j, b)

                @pl.when(jnp.logical_and(i >= 2, i - 2 < N_CHUNKS))
                def _drain():
                    scatter_wait(jd, bd)

                idx_start(i + NG + 4, (k + NG + 4) % NI)

                @pl.when(i + NG < N_CHUNKS)
                def _nxt():
                    idx_wait(jg)
                    gather_start(jg, bg)
            return carry

        lax.fori_loop(0, (N_CHUNKS + 2 + UNROLL - 1) // UNROLL, ring_step, 0)
        plsc.subcore_barrier()

        pltpu.sync_copy(agg_sh.at[pl.ds(row0, ROWS_PER_TILE), :],
                        out_hbm.at[c, pl.ds(row0, ROWS_PER_TILE), :])

        @pl.when(s == NS - 1)
        def _out_tail():
            pltpu.sync_copy(agg_sh.at[pl.ds(NS * ROWS_PER_TILE, OUT_TAIL), :],
                            out_hbm.at[c, pl.ds(NS * ROWS_PER_TILE, OUT_TAIL), :])

    return body(x, ei4, zeros)


def _tc_update(agg2, x, W, W_root, b2):
    BLK = 400

    def body(a_ref, x_ref, w_ref, wr_ref, b_ref, o_ref):
        agg = a_ref[0] + a_ref[1]
        acc = jnp.dot(agg, w_ref[...], preferred_element_type=jnp.float32)
        acc = acc + jnp.dot(x_ref[...], wr_ref[...],
                            preferred_element_type=jnp.float32)
        o_ref[...] = acc + b_ref[...]

    return pl.pallas_call(
        body,
        grid=(N_NODES // BLK,),
        in_specs=[
            pl.BlockSpec((NC, BLK, D), lambda i: (0, i, 0)),
            pl.BlockSpec((BLK, D), lambda i: (i, 0)),
            pl.BlockSpec((D, D), lambda i: (0, 0)),
            pl.BlockSpec((D, D), lambda i: (0, 0)),
            pl.BlockSpec((1, D), lambda i: (0, 0)),
        ],
        out_specs=pl.BlockSpec((BLK, D), lambda i: (i, 0)),
        out_shape=jax.ShapeDtypeStruct((N_NODES, D), jnp.float32),
    )(agg2, x, W, W_root, b2)


def kernel(x, edge_index, W, W_root, b):
    srcw = edge_index[0].reshape(NW, E_PER_W)
    dstw = edge_index[1].reshape(NW, E_PER_W)
    srcw = jnp.pad(srcw, ((0, 0), (0, E_PAD)), constant_values=0)
    dstw = jnp.pad(dstw, ((0, 0), (0, E_PAD)), constant_values=PAD_ROW)
    ei4 = jnp.stack(
        [srcw.reshape(NW, N_CHUNKS, CHUNK), dstw.reshape(NW, N_CHUNKS, CHUNK)],
        axis=2)
    zeros = jnp.zeros((N_ACC, D), jnp.float32)
    agg2 = _sc_segment_sum(x, ei4, zeros)
    return _tc_update(agg2, x, W, W_root, b.reshape(1, D))


# speedup vs baseline: 1.1466x; 1.1466x over previous
import functools

import jax
import jax.numpy as jnp
from jax import lax
from jax.experimental import pallas as pl
from jax.experimental.pallas import tpu as pltpu
from jax.experimental.pallas import tpu_sc as plsc

N_NODES = 10000
N_EDGES = 320000
D = 128

NC = 2
NS = 16
NW = NC * NS

E_PER_W = N_EDGES // NW
CHUNK = 40
N_CHUNKS = -(-E_PER_W // CHUNK)
E_PAD = N_CHUNKS * CHUNK - E_PER_W
NR = 7
NG = NR - 2
NI = 14
UNROLL = 14

PAD_ROW = N_NODES
N_ACC = 10008

ROWS_PER_TILE = 624
ZERO_TAIL = N_ACC - NS * ROWS_PER_TILE
OUT_TAIL = N_NODES - NS * ROWS_PER_TILE


def _sc_segment_sum(x, ei4, zeros):
    mesh = plsc.VectorSubcoreMesh(core_axis_name="c", subcore_axis_name="s")

    @functools.partial(
        pl.kernel,
        out_type=jax.ShapeDtypeStruct((NC, N_NODES, D), jnp.float32),
        mesh=mesh,
        scratch_types=(
            [pltpu.VMEM_SHARED((N_ACC, D), jnp.float32)]
            + [pltpu.VMEM((2, CHUNK), jnp.int32)] * NI
            + [pltpu.VMEM((CHUNK, D), jnp.float32)] * NR
            + [pltpu.SemaphoreType.DMA] * (NI + 2 * NR + 1)
        ),
        compiler_params=pltpu.CompilerParams(use_tc_tiling_on_sc=True),
    )
    def body(x_hbm, ei_hbm, z_hbm, out_hbm, agg_sh, *scr):
        ibs = scr[:NI]
        rbs = scr[NI:NI + NR]
        sis = scr[NI + NR:2 * NI + NR]
        sgs = scr[2 * NI + NR:2 * NI + 2 * NR]
        sss = scr[2 * NI + 2 * NR:2 * NI + 3 * NR]
        sz = scr[-1]
        c = lax.axis_index("c")
        s = lax.axis_index("s")
        wid = c * NS + s

        row0 = s * ROWS_PER_TILE
        pltpu.async_copy(z_hbm.at[pl.ds(row0, ROWS_PER_TILE), :],
                         agg_sh.at[pl.ds(row0, ROWS_PER_TILE), :], sz)

        @pl.when(s == NS - 1)
        def _zero_tail():
            pltpu.sync_copy(z_hbm.at[pl.ds(NS * ROWS_PER_TILE, ZERO_TAIL), :],
                            agg_sh.at[pl.ds(NS * ROWS_PER_TILE, ZERO_TAIL), :])

        def idx_start(i, j):
            @pl.when(i < N_CHUNKS)
            def _():
                pltpu.async_copy(ei_hbm.at[wid, i], ibs[j], sis[j])

        def idx_wait(j):
            pltpu.make_async_copy(ei_hbm.at[wid, 0], ibs[j], sis[j]).wait()

        def gather_start(j, b):
            pltpu.async_copy(x_hbm.at[ibs[j].at[0]], rbs[b], sgs[b])

        def gather_wait(j, b):
            pltpu.make_async_copy(x_hbm.at[ibs[j].at[0]], rbs[b],
                                  sgs[b]).wait()

        def scatter_start(j, b):
            pltpu.async_copy(rbs[b], agg_sh.at[ibs[j].at[1]], sss[b],
                             add=True)

        def scatter_wait(j, b):
            pltpu.make_async_copy(rbs[b], agg_sh.at[ibs[j].at[1]],
                                  sss[b]).wait()

        for m in range(NG + 4):
            idx_start(m, m % NI)
        for m in range(NG):
            idx_wait(m % NI)
            gather_start(m % NI, m % NR)

        pltpu.make_async_copy(z_hbm.at[pl.ds(row0, ROWS_PER_TILE), :],
                              agg_sh.at[pl.ds(row0, ROWS_PER_TILE), :],
                              sz).wait()
        plsc.subcore_barrier()

        def ring_step(q, carry):
            i0 = q * UNROLL
            for k in range(UNROLL):
                i = i0 + k
                b, j = k % NR, k % NI
                bd, jd = (k + NR - 2) % NR, (k + NI - 2) % NI
                bg, jg = (k + NG) % NR, (k + NG) % NI

                @pl.when(i < N_CHUNKS)
                def _cur():
                    gather_wait(j, b)
                    scatter_start(j, b)

                @pl.when(jnp.logical_and(i >= 2, i - 2 < N_CHUNKS))
                def _drain():
                    scatter_wait(jd, bd)

                idx_start(i + NG + 4, (k + NG + 4) % NI)

                @pl.when(i + NG < N_CHUNKS)
                def _nxt():
                    idx_wait(jg)
                    gather_start(jg, bg)
            return carry

        lax.fori_loop(0, (N_CHUNKS + 2 + UNROLL - 1) // UNROLL, ring_step, 0)
        plsc.subcore_barrier()

        pltpu.sync_copy(agg_sh.at[pl.ds(row0, ROWS_PER_TILE), :],
                        out_hbm.at[c, pl.ds(row0, ROWS_PER_TILE), :])

        @pl.when(s == NS - 1)
        def _out_tail():
            pltpu.sync_copy(agg_sh.at[pl.ds(NS * ROWS_PER_TILE, OUT_TAIL), :],
                            out_hbm.at[c, pl.ds(NS * ROWS_PER_TILE, OUT_TAIL), :])

    return body(x, ei4, zeros)


def _tc_update(agg2, x, W, W_root, b2):
    BLK = 400

    def body(a_ref, x_ref, w_ref, wr_ref, b_ref, o_ref):
        agg = a_ref[0] + a_ref[1]
        acc = jnp.dot(agg, w_ref[...], preferred_element_type=jnp.float32)
        acc = acc + jnp.dot(x_ref[...], wr_ref[...],
                            preferred_element_type=jnp.float32)
        o_ref[...] = acc + b_ref[...]

    return pl.pallas_call(
        body,
        grid=(N_NODES // BLK,),
        in_specs=[
            pl.BlockSpec((NC, BLK, D), lambda i: (0, i, 0)),
            pl.BlockSpec((BLK, D), lambda i: (i, 0)),
            pl.BlockSpec((D, D), lambda i: (0, 0)),
            pl.BlockSpec((D, D), lambda i: (0, 0)),
            pl.BlockSpec((1, D), lambda i: (0, 0)),
        ],
        out_specs=pl.BlockSpec((BLK, D), lambda i: (i, 0)),
        out_shape=jax.ShapeDtypeStruct((N_NODES, D), jnp.float32),
    )(agg2, x, W, W_root, b2)


def kernel(x, edge_index, W, W_root, b):
    srcw = edge_index[0].reshape(NW, E_PER_W)
    dstw = edge_index[1].reshape(NW, E_PER_W)
    srcw = jnp.pad(srcw, ((0, 0), (0, E_PAD)), constant_values=0)
    dstw = jnp.pad(dstw, ((0, 0), (0, E_PAD)), constant_values=PAD_ROW)
    ei4 = jnp.stack(
        [srcw.reshape(NW, N_CHUNKS, CHUNK), dstw.reshape(NW, N_CHUNKS, CHUNK)],
        axis=2)
    zeros = jnp.zeros((N_ACC, D), jnp.float32)
    agg2 = _sc_segment_sum(x, ei4, zeros)
    return _tc_update(agg2, x, W, W_root, b.reshape(1, D))


# speedup vs baseline: 1.4135x; 1.2328x over previous
import functools

import jax
import jax.numpy as jnp
from jax import lax
from jax.experimental import pallas as pl
from jax.experimental.pallas import tpu as pltpu
from jax.experimental.pallas import tpu_sc as plsc

N_NODES = 10000
N_EDGES = 320000
D = 128

NC = 2
NS = 16
NW = NC * NS

E_PER_W = N_EDGES // NW
CHUNK = 40
N_CHUNKS = -(-E_PER_W // CHUNK)
E_PAD = N_CHUNKS * CHUNK - E_PER_W
NR = 7
NG = NR - 2
NI = 14
UNROLL = 14

PAD_ROW = N_NODES
N_ACC = 10008

ROWS_PER_TILE = 624
ZERO_TAIL = N_ACC - NS * ROWS_PER_TILE
OUT_TAIL = N_NODES - NS * ROWS_PER_TILE


def _sc_segment_sum(x, src_idx, dst_idx, zeros):
    mesh = plsc.VectorSubcoreMesh(core_axis_name="c", subcore_axis_name="s")

    @functools.partial(
        pl.kernel,
        out_type=jax.ShapeDtypeStruct((NC, N_NODES, D), jnp.float32),
        mesh=mesh,
        scratch_types=(
            [pltpu.VMEM_SHARED((N_ACC, D), jnp.float32)]
            + [pltpu.VMEM((CHUNK,), jnp.int32)] * NI
            + [pltpu.VMEM((CHUNK,), jnp.int32)] * NI
            + [pltpu.VMEM((CHUNK, D), jnp.float32)] * NR
            + [pltpu.SemaphoreType.DMA] * (NI + 2 * NR + 1)
        ),
    )
    def body(x_hbm, src_hbm, dst_hbm, z_hbm, out_hbm, agg_sh, *scr):
        iss = scr[:NI]
        ids = scr[NI:2 * NI]
        rbs = scr[2 * NI:2 * NI + NR]
        sis = scr[2 * NI + NR:3 * NI + NR]
        sgs = scr[3 * NI + NR:3 * NI + 2 * NR]
        sss = scr[3 * NI + 2 * NR:3 * NI + 3 * NR]
        sz = scr[-1]
        c = lax.axis_index("c")
        s = lax.axis_index("s")
        wid = c * NS + s

        row0 = s * ROWS_PER_TILE
        pltpu.async_copy(z_hbm.at[pl.ds(row0, ROWS_PER_TILE), :],
                         agg_sh.at[pl.ds(row0, ROWS_PER_TILE), :], sz)

        @pl.when(s == NS - 1)
        def _zero_tail():
            pltpu.sync_copy(z_hbm.at[pl.ds(NS * ROWS_PER_TILE, ZERO_TAIL), :],
                            agg_sh.at[pl.ds(NS * ROWS_PER_TILE, ZERO_TAIL), :])

        ebase = wid * E_PER_W

        def idx_start(i, j):
            @pl.when(i < N_CHUNKS)
            def _():
                off = ebase + i * CHUNK
                pltpu.async_copy(src_hbm.at[pl.ds(off, CHUNK)], iss[j],
                                 sis[j])
                pltpu.async_copy(dst_hbm.at[pl.ds(off, CHUNK)], ids[j],
                                 sis[j])

        def idx_wait(j):
            pltpu.make_async_copy(src_hbm.at[pl.ds(0, CHUNK)], iss[j],
                                  sis[j]).wait()
            pltpu.make_async_copy(dst_hbm.at[pl.ds(0, CHUNK)], ids[j],
                                  sis[j]).wait()

        def gather_start(j, b):
            pltpu.async_copy(x_hbm.at[iss[j]], rbs[b], sgs[b])

        def gather_wait(j, b):
            pltpu.make_async_copy(x_hbm.at[iss[j]], rbs[b], sgs[b]).wait()

        def scatter_start(j, b):
            pltpu.async_copy(rbs[b], agg_sh.at[ids[j]], sss[b], add=True)

        def scatter_wait(j, b):
            pltpu.make_async_copy(rbs[b], agg_sh.at[ids[j]], sss[b]).wait()

        for m in range(NG + 4):
            idx_start(m, m % NI)
        for m in range(NG):
            idx_wait(m % NI)
            gather_start(m % NI, m % NR)

        pltpu.make_async_copy(z_hbm.at[pl.ds(row0, ROWS_PER_TILE), :],
                              agg_sh.at[pl.ds(row0, ROWS_PER_TILE), :],
                              sz).wait()
        plsc.subcore_barrier()

        def ring_step(q, carry):
            i0 = q * UNROLL
            for k in range(UNROLL):
                i = i0 + k
                b, j = k % NR, k % NI
                bd, jd = (k + NR - 2) % NR, (k + NI - 2) % NI
                bg, jg = (k + NG) % NR, (k + NG) % NI

                @pl.when(i < N_CHUNKS)
                def _cur():
                    gather_wait(j, b)
                    scatter_start(j, b)

                @pl.when(jnp.logical_and(i >= 2, i - 2 < N_CHUNKS))
                def _drain():
                    scatter_wait(jd, bd)

                idx_start(i + NG + 4, (k + NG + 4) % NI)

                @pl.when(i + NG < N_CHUNKS)
                def _nxt():
                    idx_wait(jg)
                    gather_start(jg, bg)
            return carry

        lax.fori_loop(0, (N_CHUNKS + 2 + UNROLL - 1) // UNROLL, ring_step, 0)
        plsc.subcore_barrier()

        pltpu.sync_copy(agg_sh.at[pl.ds(row0, ROWS_PER_TILE), :],
                        out_hbm.at[c, pl.ds(row0, ROWS_PER_TILE), :])

        @pl.when(s == NS - 1)
        def _out_tail():
            pltpu.sync_copy(agg_sh.at[pl.ds(NS * ROWS_PER_TILE, OUT_TAIL), :],
                            out_hbm.at[c, pl.ds(NS * ROWS_PER_TILE, OUT_TAIL), :])

    return body(x, src_idx, dst_idx, zeros)


def _tc_update(agg2, x, W, W_root, b2):
    BLK = 400

    def body(a_ref, x_ref, w_ref, wr_ref, b_ref, o_ref):
        agg = a_ref[0] + a_ref[1]
        acc = jnp.dot(agg, w_ref[...], preferred_element_type=jnp.float32)
        acc = acc + jnp.dot(x_ref[...], wr_ref[...],
                            preferred_element_type=jnp.float32)
        o_ref[...] = acc + b_ref[...]

    return pl.pallas_call(
        body,
        grid=(N_NODES // BLK,),
        in_specs=[
            pl.BlockSpec((NC, BLK, D), lambda i: (0, i, 0)),
            pl.BlockSpec((BLK, D), lambda i: (i, 0)),
            pl.BlockSpec((D, D), lambda i: (0, 0)),
            pl.BlockSpec((D, D), lambda i: (0, 0)),
            pl.BlockSpec((1, D), lambda i: (0, 0)),
        ],
        out_specs=pl.BlockSpec((BLK, D), lambda i: (i, 0)),
        out_shape=jax.ShapeDtypeStruct((N_NODES, D), jnp.float32),
    )(agg2, x, W, W_root, b2)


def kernel(x, edge_index, W, W_root, b):
    zeros = jnp.zeros((N_ACC, D), jnp.float32)
    agg2 = _sc_segment_sum(x, edge_index[0], edge_index[1], zeros)
    return _tc_update(agg2, x, W, W_root, b.reshape(1, D))


# speedup vs baseline: 1.5241x; 1.0782x over previous
import functools

import jax
import jax.numpy as jnp
from jax import lax
from jax.experimental import pallas as pl
from jax.experimental.pallas import tpu as pltpu
from jax.experimental.pallas import tpu_sc as plsc

N_NODES = 10000
N_EDGES = 320000
D = 128

NC = 2
NS = 16
NW = NC * NS

E_PER_W = N_EDGES // NW
CHUNK = 40
N_CHUNKS = -(-E_PER_W // CHUNK)
E_PAD = N_CHUNKS * CHUNK - E_PER_W
NR = 7
NG = NR - 2
NI = 14
UNROLL = 14

PAD_ROW = N_NODES
N_ACC = 10008

ROWS_PER_TILE = 624
ZERO_TAIL = N_ACC - NS * ROWS_PER_TILE
OUT_TAIL = N_NODES - NS * ROWS_PER_TILE


def _sc_segment_sum(x, src_idx, dst_idx, zeros):
    mesh = plsc.VectorSubcoreMesh(core_axis_name="c", subcore_axis_name="s")

    @functools.partial(
        pl.kernel,
        out_type=jax.ShapeDtypeStruct((NC, N_NODES, D), jnp.float32),
        mesh=mesh,
        scratch_types=(
            [pltpu.VMEM_SHARED((N_ACC, D), jnp.float32)]
            + [pltpu.VMEM((CHUNK,), jnp.int32)] * NI
            + [pltpu.VMEM((CHUNK,), jnp.int32)] * NI
            + [pltpu.VMEM((CHUNK, D), jnp.float32)] * NR
            + [pltpu.SemaphoreType.DMA] * (NI + 2 * NR + 1)
        ),
    )
    def body(x_hbm, src_hbm, dst_hbm, z_hbm, out_hbm, agg_sh, *scr):
        iss = scr[:NI]
        ids = scr[NI:2 * NI]
        rbs = scr[2 * NI:2 * NI + NR]
        sis = scr[2 * NI + NR:3 * NI + NR]
        sgs = scr[3 * NI + NR:3 * NI + 2 * NR]
        sss = scr[3 * NI + 2 * NR:3 * NI + 3 * NR]
        sz = scr[-1]
        c = lax.axis_index("c")
        s = lax.axis_index("s")
        wid = c * NS + s

        row0 = s * ROWS_PER_TILE
        pltpu.async_copy(z_hbm.at[pl.ds(row0, ROWS_PER_TILE), :],
                         agg_sh.at[pl.ds(row0, ROWS_PER_TILE), :], sz)

        @pl.when(s == NS - 1)
        def _zero_tail():
            pltpu.sync_copy(z_hbm.at[pl.ds(NS * ROWS_PER_TILE, ZERO_TAIL), :],
                            agg_sh.at[pl.ds(NS * ROWS_PER_TILE, ZERO_TAIL), :])

        ebase = wid * E_PER_W

        def idx_start(i, j):
            @pl.when(i < N_CHUNKS)
            def _():
                off = ebase + i * CHUNK
                pltpu.async_copy(src_hbm.at[pl.ds(off, CHUNK)], iss[j],
                                 sis[j])
                pltpu.async_copy(dst_hbm.at[pl.ds(off, CHUNK)], ids[j],
                                 sis[j])

        def idx_wait(j):
            pltpu.make_async_copy(src_hbm.at[pl.ds(0, CHUNK)], iss[j],
                                  sis[j]).wait()
            pltpu.make_async_copy(dst_hbm.at[pl.ds(0, CHUNK)], ids[j],
                                  sis[j]).wait()

        def gather_start(j, b):
            pltpu.async_copy(x_hbm.at[iss[j]], rbs[b], sgs[b])

        def gather_wait(j, b):
            pltpu.make_async_copy(x_hbm.at[iss[j]], rbs[b], sgs[b]).wait()

        def scatter_start(j, b):
            pltpu.async_copy(rbs[b], agg_sh.at[ids[j]], sss[b], add=True)

        def scatter_wait(j, b):
            pltpu.make_async_copy(rbs[b], agg_sh.at[ids[j]], sss[b]).wait()

        for m in range(NG + 4):
            idx_start(m, m % NI)
        for m in range(NG):
            idx_wait(m % NI)
            gather_start(m % NI, m % NR)

        pltpu.make_async_copy(z_hbm.at[pl.ds(row0, ROWS_PER_TILE), :],
                              agg_sh.at[pl.ds(row0, ROWS_PER_TILE), :],
                              sz).wait()
        plsc.subcore_barrier()

        def ring_step(q, carry):
            i0 = q * UNROLL
            for k in range(UNROLL):
                i = i0 + k
                b, j = k % NR, k % NI
                bd, jd = (k + NR - 2) % NR, (k + NI - 2) % NI
                bg, jg = (k + NG) % NR, (k + NG) % NI

                @pl.when(i < N_CHUNKS)
                def _cur():
                    gather_wait(j, b)
                    scatter_start(j, b)

                @pl.when(jnp.logical_and(i >= 2, i - 2 < N_CHUNKS))
                def _drain():
                    scatter_wait(jd, bd)

                idx_start(i + NG + 4, (k + NG + 4) % NI)

                @pl.when(i + NG < N_CHUNKS)
                def _nxt():
                    idx_wait(jg)
                    gather_start(jg, bg)
            return carry

        lax.fori_loop(0, (N_CHUNKS + 2 + UNROLL - 1) // UNROLL, ring_step, 0)
        plsc.subcore_barrier()

        pltpu.sync_copy(agg_sh.at[pl.ds(row0, ROWS_PER_TILE), :],
                        out_hbm.at[c, pl.ds(row0, ROWS_PER_TILE), :])

        @pl.when(s == NS - 1)
        def _out_tail():
            pltpu.sync_copy(agg_sh.at[pl.ds(NS * ROWS_PER_TILE, OUT_TAIL), :],
                            out_hbm.at[c, pl.ds(NS * ROWS_PER_TILE, OUT_TAIL), :])

    return body(x, src_idx, dst_idx, zeros)


def _tc_update(agg2, x, W, W_root, b2):
    BLK = 2000

    def body(a_ref, x_ref, w_ref, wr_ref, b_ref, o_ref):
        agg = a_ref[0] + a_ref[1]
        acc = jnp.dot(agg, w_ref[...], preferred_element_type=jnp.float32)
        acc = acc + jnp.dot(x_ref[...], wr_ref[...],
                            preferred_element_type=jnp.float32)
        o_ref[...] = acc + b_ref[...]

    return pl.pallas_call(
        body,
        grid=(N_NODES // BLK,),
        in_specs=[
            pl.BlockSpec((NC, BLK, D), lambda i: (0, i, 0)),
            pl.BlockSpec((BLK, D), lambda i: (i, 0)),
            pl.BlockSpec((D, D), lambda i: (0, 0)),
            pl.BlockSpec((D, D), lambda i: (0, 0)),
            pl.BlockSpec((1, D), lambda i: (0, 0)),
        ],
        out_specs=pl.BlockSpec((BLK, D), lambda i: (i, 0)),
        out_shape=jax.ShapeDtypeStruct((N_NODES, D), jnp.float32),
    )(agg2, x, W, W_root, b2)


def kernel(x, edge_index, W, W_root, b):
    zeros = jnp.zeros((N_ACC, D), jnp.float32)
    agg2 = _sc_segment_sum(x, edge_index[0], edge_index[1], zeros)
    return _tc_update(agg2, x, W, W_root, b.reshape(1, D))
